# Initial kernel scaffold; baseline (speedup 1.0000x reference)
import jax
import jax.numpy as jnp
from jax.experimental import pallas as pl


def kernel(x, W_gate, ln_w, ln_b, W1, W2):
    raise NotImplementedError("write your pallas kernel here")


# speedup vs baseline: 2.8987x; 2.8987x over previous
import functools

import jax
import jax.numpy as jnp
from jax.experimental import pallas as pl
from jax.experimental.pallas import tpu as pltpu

D = 2048
E = 16
K = 2
F = 1024
TBLK = 512


def _router_kernel(x_ref, wg_ref, lnw_ref, lnb_ref, w1_ref, w2_ref,
                   idx_ref, scores_ref, probs_ref, sums_ref, *, n_tokens):
    x = x_ref[...]

    mu = jnp.mean(x, axis=-1, keepdims=True)
    xc = x - mu
    var = jnp.mean(xc * xc, axis=-1, keepdims=True)
    xn = xc * jax.lax.rsqrt(var + 1e-5) * lnw_ref[...] + lnb_ref[...]

    h = jnp.dot(xn, w1_ref[...], preferred_element_type=jnp.float32)
    h = h * 0.5 * (1.0 + jax.lax.erf(h * 0.7071067811865476))
    d_logit = jnp.sum(h * w2_ref[...], axis=-1, keepdims=True)
    diff = jax.nn.sigmoid(d_logit)

    logits = jnp.dot(x, wg_ref[...], preferred_element_type=jnp.float32)
    logits = logits / (1.0 + diff)

    m = jnp.max(logits, axis=-1, keepdims=True)
    ex = jnp.exp(logits - m)
    probs = ex / jnp.sum(ex, axis=-1, keepdims=True)
    probs_ref[...] = probs

    lane = jax.lax.broadcasted_iota(jnp.int32, logits.shape, 1)
    v0 = jnp.max(logits, axis=-1, keepdims=True)
    i0 = jnp.min(jnp.where(logits == v0, lane, E), axis=-1, keepdims=True)
    masked = jnp.where(lane == i0, -jnp.inf, logits)
    v1 = jnp.max(masked, axis=-1, keepdims=True)
    i1 = jnp.min(jnp.where(masked == v1, lane, E), axis=-1, keepdims=True)
    idx_ref[...] = jnp.concatenate([i0, i1], axis=-1)

    hot0 = lane == i0
    hot1 = lane == i1
    p0 = jnp.sum(jnp.where(hot0, probs, 0.0), axis=-1, keepdims=True)
    p1 = jnp.sum(jnp.where(hot1, probs, 0.0), axis=-1, keepdims=True)
    s0 = (1.0 - p0) + p0
    s1 = (1.0 - p1) + p1
    denom = jnp.clip(s0 + s1, 1e-9, None)
    scores_ref[...] = jnp.concatenate([s0, s1], axis=-1) / denom

    imp = jnp.sum(probs, axis=0, keepdims=True)
    hard_any = (hot0 | hot1).astype(jnp.float32)
    load = jnp.sum(hard_any, axis=0, keepdims=True)
    part = jnp.concatenate(
        [imp, load, jnp.zeros((6, E), jnp.float32)], axis=0)

    @pl.when(pl.program_id(0) == 0)
    def _init():
        sums_ref[...] = jnp.zeros_like(sums_ref)

    sums_ref[...] += part

    @pl.when(pl.program_id(0) == pl.num_programs(0) - 1)
    def _fini():
        sums_ref[...] *= (1.0 / n_tokens)


def kernel(x, W_gate, ln_w, ln_b, W1, W2):
    B, S, _ = x.shape
    n = B * S
    x2 = x.reshape(n, D)
    grid = (n // TBLK,)

    out = pl.pallas_call(
        functools.partial(_router_kernel, n_tokens=float(n)),
        grid=grid,
        in_specs=[
            pl.BlockSpec((TBLK, D), lambda i: (i, 0)),
            pl.BlockSpec((D, E), lambda i: (0, 0)),
            pl.BlockSpec((1, D), lambda i: (0, 0)),
            pl.BlockSpec((1, D), lambda i: (0, 0)),
            pl.BlockSpec((D, F), lambda i: (0, 0)),
            pl.BlockSpec((1, F), lambda i: (0, 0)),
        ],
        out_specs=[
            pl.BlockSpec((TBLK, K), lambda i: (i, 0)),
            pl.BlockSpec((TBLK, K), lambda i: (i, 0)),
            pl.BlockSpec((TBLK, E), lambda i: (i, 0)),
            pl.BlockSpec((8, E), lambda i: (0, 0)),
        ],
        out_shape=[
            jax.ShapeDtypeStruct((n, K), jnp.int32),
            jax.ShapeDtypeStruct((n, K), jnp.float32),
            jax.ShapeDtypeStruct((n, E), jnp.float32),
            jax.ShapeDtypeStruct((8, E), jnp.float32),
        ],
    )(x2, W_gate, ln_w.reshape(1, D), ln_b.reshape(1, D), W1,
      W2.reshape(1, F))

    idx, scores, probs, sums = out
    return (idx.reshape(B, S, K), scores.reshape(B, S, K),
            probs.reshape(B, S, E), sums[0], sums[1])


# speedup vs baseline: 2.9153x; 1.0057x over previous
import functools

import jax
import jax.numpy as jnp
from jax.experimental import pallas as pl
from jax.experimental.pallas import tpu as pltpu

D = 2048
E = 16
K = 2
F = 1024
TBLK = 512


def _router_kernel(x_ref, wg_ref, lnw_ref, lnb_ref, w1_ref, w2_ref,
                   idx_ref, scores_ref, probs_ref, sums_ref, *, n_tokens):
    x = x_ref[...]

    mu = jnp.mean(x, axis=-1, keepdims=True)
    ex2 = jnp.mean(x * x, axis=-1, keepdims=True)
    var = ex2 - mu * mu
    r = jax.lax.rsqrt(var + 1e-5)
    xn = (x - mu) * r * lnw_ref[...] + lnb_ref[...]

    h = jnp.dot(xn.astype(jnp.bfloat16), w1_ref[...],
                preferred_element_type=jnp.float32)
    h = h * 0.5 * (1.0 + jax.lax.erf(h * 0.7071067811865476))
    d_logit = jnp.sum(h * w2_ref[...], axis=-1, keepdims=True)
    diff = jax.nn.sigmoid(d_logit)

    logits = jnp.dot(x, wg_ref[...], preferred_element_type=jnp.float32)
    logits = logits / (1.0 + diff)

    m = jnp.max(logits, axis=-1, keepdims=True)
    ex = jnp.exp(logits - m)
    probs = ex / jnp.sum(ex, axis=-1, keepdims=True)
    probs_ref[...] = probs

    lane = jax.lax.broadcasted_iota(jnp.int32, logits.shape, 1)
    v0 = jnp.max(logits, axis=-1, keepdims=True)
    i0 = jnp.min(jnp.where(logits == v0, lane, E), axis=-1, keepdims=True)
    masked = jnp.where(lane == i0, -jnp.inf, logits)
    v1 = jnp.max(masked, axis=-1, keepdims=True)
    i1 = jnp.min(jnp.where(masked == v1, lane, E), axis=-1, keepdims=True)
    idx_ref[...] = jnp.concatenate([i0, i1], axis=-1)

    hot0 = lane == i0
    hot1 = lane == i1
    p0 = jnp.sum(jnp.where(hot0, probs, 0.0), axis=-1, keepdims=True)
    p1 = jnp.sum(jnp.where(hot1, probs, 0.0), axis=-1, keepdims=True)
    s0 = (1.0 - p0) + p0
    s1 = (1.0 - p1) + p1
    denom = jnp.clip(s0 + s1, 1e-9, None)
    scores_ref[...] = jnp.concatenate([s0, s1], axis=-1) / denom

    imp = jnp.sum(probs, axis=0, keepdims=True)
    hard_any = (hot0 | hot1).astype(jnp.float32)
    load = jnp.sum(hard_any, axis=0, keepdims=True)
    part = jnp.concatenate(
        [imp, load, jnp.zeros((6, E), jnp.float32)], axis=0)

    @pl.when(pl.program_id(0) == 0)
    def _init():
        sums_ref[...] = jnp.zeros_like(sums_ref)

    sums_ref[...] += part

    @pl.when(pl.program_id(0) == pl.num_programs(0) - 1)
    def _fini():
        sums_ref[...] *= (1.0 / n_tokens)


def kernel(x, W_gate, ln_w, ln_b, W1, W2):
    B, S, _ = x.shape
    n = B * S
    x2 = x.reshape(n, D)
    grid = (n // TBLK,)

    out = pl.pallas_call(
        functools.partial(_router_kernel, n_tokens=float(n)),
        grid=grid,
        in_specs=[
            pl.BlockSpec((TBLK, D), lambda i: (i, 0)),
            pl.BlockSpec((D, E), lambda i: (0, 0)),
            pl.BlockSpec((1, D), lambda i: (0, 0)),
            pl.BlockSpec((1, D), lambda i: (0, 0)),
            pl.BlockSpec((D, F), lambda i: (0, 0)),
            pl.BlockSpec((1, F), lambda i: (0, 0)),
        ],
        out_specs=[
            pl.BlockSpec((TBLK, K), lambda i: (i, 0)),
            pl.BlockSpec((TBLK, K), lambda i: (i, 0)),
            pl.BlockSpec((TBLK, E), lambda i: (i, 0)),
            pl.BlockSpec((8, E), lambda i: (0, 0)),
        ],
        out_shape=[
            jax.ShapeDtypeStruct((n, K), jnp.int32),
            jax.ShapeDtypeStruct((n, K), jnp.float32),
            jax.ShapeDtypeStruct((n, E), jnp.float32),
            jax.ShapeDtypeStruct((8, E), jnp.float32),
        ],
    )(x2, W_gate, ln_w.reshape(1, D), ln_b.reshape(1, D),
      W1.astype(jnp.bfloat16), W2.reshape(1, F))

    idx, scores, probs, sums = out
    return (idx.reshape(B, S, K), scores.reshape(B, S, K),
            probs.reshape(B, S, E), sums[0], sums[1])


# speedup vs baseline: 3.5572x; 1.2202x over previous
import functools

import jax
import jax.numpy as jnp
from jax.experimental import pallas as pl
from jax.experimental.pallas import tpu as pltpu

D = 2048
E = 16
K = 2
F = 1024
GCOLS = 32
TBLK = 1024


def _router_kernel(x_ref, wg_ref, w1_ref, c1_ref, cb_ref, w2_ref, ones_ref,
                   idx_ref, scores_ref, probs_ref, sums_ref, *, n_tokens):
    x = x_ref[...]
    xb = x.astype(jnp.bfloat16)

    g = jnp.dot(x, wg_ref[...], preferred_element_type=jnp.float32)
    logits_raw = g[:, :E]
    mu = g[:, E:E + 1] * (1.0 / D)

    sumsq = jnp.dot(xb * xb, ones_ref[...],
                    preferred_element_type=jnp.float32)[:, :1]
    var = sumsq * (1.0 / D) - mu * mu
    r = jax.lax.rsqrt(var + 1e-5)

    y = jnp.dot(xb, w1_ref[...],
                preferred_element_type=jnp.float32).astype(jnp.bfloat16)
    h = y * r.astype(jnp.bfloat16) - (r * mu).astype(jnp.bfloat16) * c1_ref[...] + cb_ref[...]

    hh = jnp.bfloat16(0.5) * h
    h = hh + hh * jax.lax.erf(h * jnp.bfloat16(0.7071067811865476))

    d_logit = jnp.sum(h * w2_ref[...], axis=-1, keepdims=True)
    diff = jax.nn.sigmoid(d_logit.astype(jnp.float32))

    lg_t = jnp.transpose(logits_raw)
    diff_t = jnp.transpose(diff)
    logits = lg_t / (1.0 + diff_t)

    m = jnp.max(logits, axis=0, keepdims=True)
    ex = jnp.exp(logits - m)
    probs = ex / jnp.sum(ex, axis=0, keepdims=True)
    probs_ref[...] = probs

    srow = jax.lax.broadcasted_iota(jnp.int32, logits.shape, 0)
    v0 = jnp.max(logits, axis=0, keepdims=True)
    i0 = jnp.min(jnp.where(logits == v0, srow, E), axis=0, keepdims=True)
    masked = jnp.where(srow == i0, -jnp.inf, logits)
    v1 = jnp.max(masked, axis=0, keepdims=True)
    i1 = jnp.min(jnp.where(masked == v1, srow, E), axis=0, keepdims=True)
    idx_ref[...] = jnp.concatenate([i0, i1], axis=0)

    hot0 = srow == i0
    hot1 = srow == i1
    p0 = jnp.sum(jnp.where(hot0, probs, 0.0), axis=0, keepdims=True)
    p1 = jnp.sum(jnp.where(hot1, probs, 0.0), axis=0, keepdims=True)
    s0 = (1.0 - p0) + p0
    s1 = (1.0 - p1) + p1
    denom = jnp.clip(s0 + s1, 1e-9, None)
    scores_ref[...] = jnp.concatenate([s0, s1], axis=0) / denom

    imp = jnp.sum(probs, axis=1, keepdims=True)
    hard_any = (hot0 | hot1).astype(jnp.float32)
    load = jnp.sum(hard_any, axis=1, keepdims=True)
    part = jnp.concatenate(
        [imp, load, jnp.zeros((E, 126), jnp.float32)], axis=1)

    @pl.when(pl.program_id(0) == 0)
    def _init():
        sums_ref[...] = jnp.zeros_like(sums_ref)

    sums_ref[...] += part

    @pl.when(pl.program_id(0) == pl.num_programs(0) - 1)
    def _fini():
        sums_ref[...] *= (1.0 / n_tokens)


def kernel(x, W_gate, ln_w, ln_b, W1, W2):
    B, S, _ = x.shape
    n = B * S
    x2 = x.reshape(n, D)
    grid = (n // TBLK,)

    wg_ext = jnp.concatenate(
        [W_gate, jnp.ones((D, 1), jnp.float32),
         jnp.zeros((D, GCOLS - E - 1), jnp.float32)], axis=1)
    w1b = (W1 * ln_w[:, None]).astype(jnp.bfloat16)
    c1 = (ln_w @ W1).reshape(1, F).astype(jnp.bfloat16)
    cb = (ln_b @ W1).reshape(1, F).astype(jnp.bfloat16)
    ones_col = jnp.ones((D, 1), jnp.bfloat16)

    out = pl.pallas_call(
        functools.partial(_router_kernel, n_tokens=float(n)),
        grid=grid,
        in_specs=[
            pl.BlockSpec((TBLK, D), lambda i: (i, 0)),
            pl.BlockSpec((D, GCOLS), lambda i: (0, 0)),
            pl.BlockSpec((D, F), lambda i: (0, 0)),
            pl.BlockSpec((1, F), lambda i: (0, 0)),
            pl.BlockSpec((1, F), lambda i: (0, 0)),
            pl.BlockSpec((1, F), lambda i: (0, 0)),
            pl.BlockSpec((D, 1), lambda i: (0, 0)),
        ],
        out_specs=[
            pl.BlockSpec((K, TBLK), lambda i: (0, i)),
            pl.BlockSpec((K, TBLK), lambda i: (0, i)),
            pl.BlockSpec((E, TBLK), lambda i: (0, i)),
            pl.BlockSpec((E, 128), lambda i: (0, 0)),
        ],
        out_shape=[
            jax.ShapeDtypeStruct((K, n), jnp.int32),
            jax.ShapeDtypeStruct((K, n), jnp.float32),
            jax.ShapeDtypeStruct((E, n), jnp.float32),
            jax.ShapeDtypeStruct((E, 128), jnp.float32),
        ],
    )(x2, wg_ext, w1b, c1, cb, W2.reshape(1, F).astype(jnp.bfloat16), ones_col)

    idx_t, scores_t, probs_t, sums = out
    return (idx_t.T.reshape(B, S, K), scores_t.T.reshape(B, S, K),
            probs_t.T.reshape(B, S, E), sums[:, 0], sums[:, 1])
